# baseline (speedup 1.0000x reference)
import jax
import jax.numpy as jnp
from jax import lax
from jax.experimental import pallas as pl
from jax.experimental.pallas import tpu as pltpu
from jax.experimental.pallas import tpu_sc as plsc

HIDDEN = 768
NCHUNK = HIDDEN // 16
EPS = 1e-12

NC = 2
NS = 16
NW = NC * NS

TOKENS = 8192
T_PER_W = TOKENS // NW
G = 16
TB = 4
NSUB = T_PER_W // G
NSUB2 = NSUB // 2
SEQ = 2048


def _body(word_hbm, comb_hbm, pos_hbm, tok_hbm, seg_hbm, gamma_hbm, beta_hbm,
          out_hbm,
          idxall, cidxall, wbuf0, wbuf1, cbuf0, cbuf1, pbuf0, pbuf1,
          gbuf, bbuf, sw0, sw1, sc0, sc1, sp0, sp1, so0, so1):
    wid = lax.axis_index("s") * NC + lax.axis_index("c")
    wbase = wid * T_PER_W
    s0 = (wid % (SEQ // T_PER_W)) * T_PER_W

    pltpu.sync_copy(gamma_hbm, gbuf)
    pltpu.sync_copy(beta_hbm, bbuf)
    pltpu.sync_copy(tok_hbm.at[pl.ds(wbase, T_PER_W)], idxall)
    pltpu.sync_copy(seg_hbm.at[pl.ds(wbase, T_PER_W)], cidxall)

    def issue(i, wb, cb, pb, sw, sc, sp):
        sbase = s0 + i * G
        pltpu.async_copy(word_hbm.at[idxall.at[pl.ds(i * G, G)]], wb, sw)
        pltpu.async_copy(comb_hbm.at[cidxall.at[pl.ds(i * G, G)]], cb, sc)
        pltpu.async_copy(pos_hbm.at[pl.ds(sbase, G)], pb, sp)

    def wait_gathers(wb, cb, pb, sw, sc, sp):
        pltpu.make_async_copy(word_hbm.at[idxall.at[pl.ds(0, G)]], wb, sw).wait()
        pltpu.make_async_copy(comb_hbm.at[cidxall.at[pl.ds(0, G)]], cb, sc).wait()
        pltpu.make_async_copy(pos_hbm.at[pl.ds(s0, G)], pb, sp).wait()

    def scatter(i, wb, so):
        pltpu.async_copy(wb, out_hbm.at[pl.ds(wbase + i * G, G)], so)

    def wait_scatter(wb, so):
        pltpu.make_async_copy(wb, out_hbm.at[pl.ds(wbase, G)], so).wait()

    def allsum(x):
        for sh in (8, 4, 2, 1):
            idx = lax.iota(jnp.int32, 16) ^ sh
            x = x + x.at[idx].get(mode="promise_in_bounds")
        return x

    def compute(wb, cb, pb):
        def tgroup(tg, c2):
            t0 = tg * TB
            stats = []
            for k in range(TB):
                t = t0 + k
                acc_s = jnp.zeros((16,), jnp.float32)
                acc_q = jnp.zeros((16,), jnp.float32)
                for j in range(NCHUNK):
                    sl = pl.ds(j * 16, 16)
                    x = wb[t, sl] + pb[t, sl] + cb[t, sl]
                    wb[t, sl] = x
                    acc_s = acc_s + x
                    acc_q = acc_q + x * x
                muv = allsum(acc_s) * (1.0 / HIDDEN)
                v = allsum(acc_q) * (1.0 / HIDDEN) - muv * muv + EPS
                bits = plsc.bitcast(v, jnp.int32)
                y = plsc.bitcast(jnp.int32(0x5F3759DF) - (bits >> 1),
                                 jnp.float32)
                half = v * 0.5
                for _ in range(4):
                    y = y * (1.5 - half * y * y)
                stats.append((muv, y))
            for j in range(NCHUNK):
                sl = pl.ds(j * 16, 16)
                g = gbuf[sl]
                b = bbuf[sl]
                xs = [wb[t0 + k, sl] for k in range(TB)]
                for k in range(TB):
                    muv, y = stats[k]
                    xs[k] = (xs[k] - muv) * y * g + b
                for k in range(TB):
                    wb[t0 + k, sl] = xs[k]
            return c2

        lax.fori_loop(0, G // TB, tgroup, 0)

    issue(0, wbuf0, cbuf0, pbuf0, sw0, sc0, sp0)

    def step(i2, carry):
        ie = 2 * i2
        io = ie + 1

        @pl.when(i2 > 0)
        def _():
            wait_scatter(wbuf1, so1)

        issue(io, wbuf1, cbuf1, pbuf1, sw1, sc1, sp1)
        wait_gathers(wbuf0, cbuf0, pbuf0, sw0, sc0, sp0)
        compute(wbuf0, cbuf0, pbuf0)
        scatter(ie, wbuf0, so0)

        @pl.when(i2 + 1 < NSUB2)
        def _():
            wait_scatter(wbuf0, so0)
            issue(ie + 2, wbuf0, cbuf0, pbuf0, sw0, sc0, sp0)

        wait_gathers(wbuf1, cbuf1, pbuf1, sw1, sc1, sp1)
        compute(wbuf1, cbuf1, pbuf1)
        scatter(io, wbuf1, so1)
        return carry

    lax.fori_loop(0, NSUB2, step, 0)
    wait_scatter(wbuf0, so0)
    wait_scatter(wbuf1, so1)


@jax.jit
def _run(word_emb, comb, pos_emb, tok, seg, gamma, beta):
    mesh = plsc.VectorSubcoreMesh(core_axis_name="c", subcore_axis_name="s",
                                  num_cores=NC, num_subcores=NS)
    return pl.kernel(
        _body,
        out_type=jax.ShapeDtypeStruct((TOKENS, HIDDEN), jnp.float32),
        mesh=mesh,
        compiler_params=pltpu.CompilerParams(needs_layout_passes=False),
        scratch_types=[
            pltpu.VMEM((T_PER_W,), jnp.int32),
            pltpu.VMEM((T_PER_W,), jnp.int32),
            pltpu.VMEM((G, HIDDEN), jnp.float32),
            pltpu.VMEM((G, HIDDEN), jnp.float32),
            pltpu.VMEM((G, HIDDEN), jnp.float32),
            pltpu.VMEM((G, HIDDEN), jnp.float32),
            pltpu.VMEM((G, HIDDEN), jnp.float32),
            pltpu.VMEM((G, HIDDEN), jnp.float32),
            pltpu.VMEM((HIDDEN,), jnp.float32),
            pltpu.VMEM((HIDDEN,), jnp.float32),
            pltpu.SemaphoreType.DMA,
            pltpu.SemaphoreType.DMA,
            pltpu.SemaphoreType.DMA,
            pltpu.SemaphoreType.DMA,
            pltpu.SemaphoreType.DMA,
            pltpu.SemaphoreType.DMA,
            pltpu.SemaphoreType.DMA,
            pltpu.SemaphoreType.DMA,
        ],
    )(word_emb, comb, pos_emb, tok, seg, gamma, beta)


def kernel(token_ids, segment_ids, question_type, word_emb, pos_emb, tt_emb,
           st_emb, gamma, beta):
    B, S = token_ids.shape
    comb = st_emb + tt_emb[(jnp.arange(st_emb.shape[0]) > 0).astype(jnp.int32)]
    tok = token_ids.reshape(-1).astype(jnp.int32)
    seg = segment_ids.reshape(-1).astype(jnp.int32)
    out = _run(word_emb, comb, pos_emb, tok, seg, gamma, beta)
    return out.reshape(B, S, HIDDEN)


# speedup vs baseline: 1.7323x; 1.7323x over previous
import jax
import jax.numpy as jnp
from jax import lax
from jax.experimental import pallas as pl
from jax.experimental.pallas import tpu as pltpu
from jax.experimental.pallas import tpu_sc as plsc

HIDDEN = 768
NCHUNK = HIDDEN // 16
EPS = 1e-12

NC = 2
NS = 16
NW = NC * NS

TOKENS = 8192
T_PER_W = TOKENS // NW
G = 16
TB = 4
NSUB = T_PER_W // G
NSUB2 = NSUB // 2
SEQ = 2048


def _body(word_hbm, comb_hbm, pos_hbm, tok_hbm, seg_hbm, gamma_hbm, beta_hbm,
          out_hbm,
          idxall, cidxall, wbuf0, wbuf1, cbuf0, cbuf1, pbuf0, pbuf1,
          obuf0, obuf1, gbuf, bbuf, sw0, sw1, sc0, sc1, sp0, sp1, so0, so1):
    wid = lax.axis_index("s") * NC + lax.axis_index("c")
    wbase = wid * T_PER_W
    s0 = (wid % (SEQ // T_PER_W)) * T_PER_W

    pltpu.sync_copy(tok_hbm.at[pl.ds(wbase, T_PER_W)], idxall)
    pltpu.sync_copy(seg_hbm.at[pl.ds(wbase, T_PER_W)], cidxall)

    def issue(i, wb, cb, pb, sw, sc, sp):
        sbase = s0 + i * G
        pltpu.async_copy(word_hbm.at[idxall.at[pl.ds(i * G, G)]], wb, sw)
        pltpu.async_copy(comb_hbm.at[cidxall.at[pl.ds(i * G, G)]], cb, sc)
        pltpu.async_copy(pos_hbm.at[pl.ds(sbase, G)], pb, sp)

    def wait_gathers(wb, cb, pb, sw, sc, sp):
        pltpu.make_async_copy(word_hbm.at[idxall.at[pl.ds(0, G)]], wb, sw).wait()
        pltpu.make_async_copy(comb_hbm.at[cidxall.at[pl.ds(0, G)]], cb, sc).wait()
        pltpu.make_async_copy(pos_hbm.at[pl.ds(s0, G)], pb, sp).wait()

    def scatter(i, wb, so):
        pltpu.async_copy(wb, out_hbm.at[pl.ds(wbase + i * G, G)], so)

    def wait_scatter(wb, so):
        pltpu.make_async_copy(wb, out_hbm.at[pl.ds(wbase, G)], so).wait()

    def allsum(x):
        for sh in (8, 4, 2, 1):
            idx = lax.iota(jnp.int32, 16) ^ sh
            x = x + x.at[idx].get(mode="promise_in_bounds")
        return x

    def pass1(wb, cb, pb, t):
        acc_s = jnp.zeros((16,), jnp.float32)
        acc_q = jnp.zeros((16,), jnp.float32)
        for j in range(NCHUNK):
            sl = pl.ds(j * 16, 16)
            x = wb[t, sl] + pb[t, sl] + cb[t, sl]
            wb[t, sl] = x
            acc_s = acc_s + x
            acc_q = acc_q + x * x
        muv = allsum(acc_s) * (1.0 / HIDDEN)
        v = allsum(acc_q) * (1.0 / HIDDEN) - muv * muv + EPS
        bits = plsc.bitcast(v, jnp.int32)
        y = plsc.bitcast(jnp.int32(0x5F3759DF) - (bits >> 1), jnp.float32)
        half = v * 0.5
        for _ in range(3):
            y = y * (1.5 - half * y * y)
        return muv, y

    def pass2(wb, ob, t, muv, y):
        for j in range(NCHUNK):
            sl = pl.ds(j * 16, 16)
            ob[t, sl] = (wb[t, sl] - muv) * y * gbuf[sl] + bbuf[sl]

    def compute(wb, cb, pb, ob):
        def pipe(t, carry):
            muv, y = carry
            pass2(wb, ob, t - 1, muv, y)
            return pass1(wb, cb, pb, t)

        muv, y = lax.fori_loop(1, G, pipe, pass1(wb, cb, pb, 0))
        pass2(wb, ob, G - 1, muv, y)

    issue(0, wbuf0, cbuf0, pbuf0, sw0, sc0, sp0)
    pltpu.sync_copy(gamma_hbm, gbuf)
    pltpu.sync_copy(beta_hbm, bbuf)

    def step(i2, carry):
        ie = 2 * i2
        io = ie + 1

        issue(io, wbuf1, cbuf1, pbuf1, sw1, sc1, sp1)
        wait_gathers(wbuf0, cbuf0, pbuf0, sw0, sc0, sp0)

        @pl.when(i2 > 0)
        def _():
            wait_scatter(obuf0, so0)

        compute(wbuf0, cbuf0, pbuf0, obuf0)
        scatter(ie, obuf0, so0)

        @pl.when(i2 + 1 < NSUB2)
        def _():
            issue(ie + 2, wbuf0, cbuf0, pbuf0, sw0, sc0, sp0)

        wait_gathers(wbuf1, cbuf1, pbuf1, sw1, sc1, sp1)

        @pl.when(i2 > 0)
        def _():
            wait_scatter(obuf1, so1)

        compute(wbuf1, cbuf1, pbuf1, obuf1)
        scatter(io, obuf1, so1)
        return carry

    lax.fori_loop(0, NSUB2, step, 0)
    wait_scatter(obuf0, so0)
    wait_scatter(obuf1, so1)


@jax.jit
def _run(word_emb, comb, pos_emb, tok, seg, gamma, beta):
    mesh = plsc.VectorSubcoreMesh(core_axis_name="c", subcore_axis_name="s",
                                  num_cores=NC, num_subcores=NS)
    return pl.kernel(
        _body,
        out_type=jax.ShapeDtypeStruct((TOKENS, HIDDEN), jnp.float32),
        mesh=mesh,
        compiler_params=pltpu.CompilerParams(needs_layout_passes=False),
        scratch_types=[
            pltpu.VMEM((T_PER_W,), jnp.int32),
            pltpu.VMEM((T_PER_W,), jnp.int32),
            pltpu.VMEM((G, HIDDEN), jnp.float32),
            pltpu.VMEM((G, HIDDEN), jnp.float32),
            pltpu.VMEM((G, HIDDEN), jnp.float32),
            pltpu.VMEM((G, HIDDEN), jnp.float32),
            pltpu.VMEM((G, HIDDEN), jnp.float32),
            pltpu.VMEM((G, HIDDEN), jnp.float32),
            pltpu.VMEM((G, HIDDEN), jnp.float32),
            pltpu.VMEM((G, HIDDEN), jnp.float32),
            pltpu.VMEM((HIDDEN,), jnp.float32),
            pltpu.VMEM((HIDDEN,), jnp.float32),
            pltpu.SemaphoreType.DMA,
            pltpu.SemaphoreType.DMA,
            pltpu.SemaphoreType.DMA,
            pltpu.SemaphoreType.DMA,
            pltpu.SemaphoreType.DMA,
            pltpu.SemaphoreType.DMA,
            pltpu.SemaphoreType.DMA,
            pltpu.SemaphoreType.DMA,
        ],
    )(word_emb, comb, pos_emb, tok, seg, gamma, beta)


def kernel(token_ids, segment_ids, question_type, word_emb, pos_emb, tt_emb,
           st_emb, gamma, beta):
    B, S = token_ids.shape
    comb = st_emb + tt_emb[(jnp.arange(st_emb.shape[0]) > 0).astype(jnp.int32)]
    tok = token_ids.reshape(-1).astype(jnp.int32)
    seg = segment_ids.reshape(-1).astype(jnp.int32)
    out = _run(word_emb, comb, pos_emb, tok, seg, gamma, beta)
    return out.reshape(B, S, HIDDEN)
